# baseline (speedup 1.0000x reference)
import jax
import jax.numpy as jnp
from jax.experimental import pallas as pl


def _identity_kernel(x_ref, o_ref):
    o_ref[...] = x_ref[...]


def kernel(unique_nodes, selected_node, selected_edge_idxs, selected_delta_time,
           selected_weight, node_features, edge_features, time_w, time_b,
           fc1_W, fc1_b, fc2_W, fc2_b, fc1s_W, fc1s_b, fc2s_W, fc2s_b,
           upd_W, upd_b):
    src = jnp.take(node_features, unique_nodes, axis=0)
    h = jax.nn.relu(src @ fc1s_W + fc1s_b)
    src_emb = h @ fc2s_W + fc2s_b
    nf = jnp.take(node_features, selected_node, axis=0)
    ef = jnp.take(edge_features, selected_edge_idxs, axis=0)
    te = jnp.cos(selected_delta_time[..., None] * time_w + time_b)
    neigh = jnp.concatenate([nf, ef, te], axis=-1)
    h = jax.nn.relu(neigh @ fc1_W + fc1_b)
    neigh = h @ fc2_W + fc2_b
    ws = jnp.sum(selected_weight, axis=1)
    ws = jnp.where(ws < 1e-06, 1e-06, ws)
    w = selected_weight / ws[:, None]
    agg = jnp.sum(neigh * w[:, :, None], axis=1)
    emb = jnp.concatenate([src_emb, agg], axis=1) @ upd_W + upd_b
    emb = pl.pallas_call(
        _identity_kernel,
        out_shape=jax.ShapeDtypeStruct(emb.shape, emb.dtype),
    )(emb)
    new_node_features = node_features.at[unique_nodes].set(emb)
    return emb, new_node_features


# speedup vs baseline: 5.2826x; 5.2826x over previous
import jax
import jax.numpy as jnp
from jax import lax
from jax.experimental import pallas as pl
from jax.experimental.pallas import tpu as pltpu
from jax.experimental.pallas import tpu_sc as plsc

N_NODES = 100000
D = 256
D_EDGE = 16
D_TIME = 100
B = 8192
K = 32
BK = B * K

NC, NS = 2, 16
NW = NC * NS
CH = 128
RPW = BK // NW
NCHUNK = RPW // CH
SPW = B // NW

BS = 128
GRID = B // BS
TBLK = 1568


def _sc_gather_body(nodes, selnode, uniq,
                    nf_out, src_out,
                    nidx, uidx, nr0, nr1,
                    gs0, gs1, ws0, ws1):
    wid = lax.axis_index("s") * NC + lax.axis_index("c")
    base = wid * RPW
    nrows = (nr0, nr1)
    gsem = (gs0, gs1)
    wsem = (ws0, ws1)

    pltpu.sync_copy(selnode.at[pl.ds(base, RPW)], nidx)

    @pl.loop(0, NCHUNK, step=2)
    def _pair(cc):
        @pl.when(cc > 0)
        def _():
            for s in range(2):
                pltpu.make_async_copy(
                    nrows[s], nf_out.at[pl.ds(base, CH)], wsem[s]).wait()
        gn = []
        for s in range(2):
            c = cc + s
            gn.append(pltpu.async_copy(
                nodes.at[nidx.at[pl.ds(c * CH, CH)]], nrows[s], gsem[s]))
        for s in range(2):
            c = cc + s
            gn[s].wait()
            pltpu.async_copy(nrows[s], nf_out.at[pl.ds(base + c * CH, CH)],
                             wsem[s])

    for s in range(2):
        pltpu.make_async_copy(nrows[s], nf_out.at[pl.ds(base, CH)],
                              wsem[s]).wait()

    sbase = wid * SPW
    pltpu.sync_copy(uniq.at[pl.ds(sbase, SPW)], uidx)
    for t in range(2):
        pltpu.async_copy(nodes.at[uidx.at[pl.ds(t * CH, CH)]], nrows[t],
                         gsem[t]).wait()
        pltpu.sync_copy(nrows[t], src_out.at[pl.ds(sbase + t * CH, CH)])


def _sc_gather(node_features, selnode_flat, unique_nodes):
    mesh = plsc.VectorSubcoreMesh(core_axis_name="c", subcore_axis_name="s")
    return pl.kernel(
        _sc_gather_body,
        out_type=(
            jax.ShapeDtypeStruct((BK, D), jnp.float32),
            jax.ShapeDtypeStruct((B, D), jnp.float32),
        ),
        mesh=mesh,
        scratch_types=(
            pltpu.VMEM((RPW,), jnp.int32),
            pltpu.VMEM((SPW,), jnp.int32),
            pltpu.VMEM((CH, D), jnp.float32),
            pltpu.VMEM((CH, D), jnp.float32),
        ) + (pltpu.SemaphoreType.DMA,) * 4,
        name="sc_gather",
    )(node_features, selnode_flat, unique_nodes)


def _sc_gather_edges_body(edges, seledge, ef_out, eidx, er0, er1,
                          ge0, ge1, we0, we1):
    wid = lax.axis_index("s") * NC + lax.axis_index("c")
    base = wid * RPW
    erows = (er0, er1)
    esem = (ge0, ge1)
    wesem = (we0, we1)

    pltpu.sync_copy(seledge.at[pl.ds(base, RPW)], eidx)

    @pl.loop(0, NCHUNK, step=2)
    def _pair(cc):
        @pl.when(cc > 0)
        def _():
            for s in range(2):
                pltpu.make_async_copy(
                    erows[s], ef_out.at[pl.ds(base, CH)], wesem[s]).wait()
        ge = []
        for s in range(2):
            c = cc + s
            ge.append(pltpu.async_copy(
                edges.at[eidx.at[pl.ds(c * CH, CH)]], erows[s], esem[s]))
        for s in range(2):
            c = cc + s
            ge[s].wait()
            pltpu.async_copy(erows[s], ef_out.at[pl.ds(base + c * CH, CH)],
                             wesem[s])

    for s in range(2):
        pltpu.make_async_copy(erows[s], ef_out.at[pl.ds(base, CH)],
                              wesem[s]).wait()


def _sc_gather_edges(edge_features, seledge_flat):
    mesh = plsc.VectorSubcoreMesh(core_axis_name="c", subcore_axis_name="s")
    return pl.kernel(
        _sc_gather_edges_body,
        out_type=jax.ShapeDtypeStruct((BK, D_EDGE), jnp.float32),
        mesh=mesh,
        scratch_types=(
            pltpu.VMEM((RPW,), jnp.int32),
            pltpu.VMEM((CH, D_EDGE), jnp.float32),
            pltpu.VMEM((CH, D_EDGE), jnp.float32),
        ) + (pltpu.SemaphoreType.DMA,) * 4,
        compiler_params=pltpu.CompilerParams(use_tc_tiling_on_sc=False),
        name="sc_gather_edges",
    )(edge_features, seledge_flat)


def _tc_mlp_body(nf, ef, dt, wflat, w2d, srcrows, ucol, urow, tbl_in,
                 w1n, w1e, w1t, b1, w2, b2, w1s, b1s, w2s, b2s,
                 updA, updB, updb, twp, tbp,
                 emb_out, win_out, tbl_out):
    tbl_out[...] = tbl_in[...]

    te = jnp.cos(dt[...] * twp[...] + tbp[...])
    h = (jnp.dot(nf[...], w1n[...], preferred_element_type=jnp.float32)
         + jnp.dot(ef[...], w1e[...], preferred_element_type=jnp.float32)
         + jnp.dot(te, w1t[...], preferred_element_type=jnp.float32)
         + b1[...])
    h = jnp.maximum(h, 0.0)
    h2 = jnp.dot(h, w2[...], preferred_element_type=jnp.float32) + b2[...]
    sc = h2 * wflat[...]
    agg = jnp.sum(sc.reshape(BS, K, D), axis=1)
    ws = jnp.sum(w2d[...], axis=1, keepdims=True)
    ws = jnp.where(ws < 1e-6, 1e-6, ws)
    agg = agg / ws

    hs = jnp.maximum(
        jnp.dot(srcrows[...], w1s[...], preferred_element_type=jnp.float32)
        + b1s[...], 0.0)
    semb = jnp.dot(hs, w2s[...], preferred_element_type=jnp.float32) + b2s[...]
    emb_out[...] = (jnp.dot(semb, updA[...], preferred_element_type=jnp.float32)
                    + jnp.dot(agg, updB[...], preferred_element_type=jnp.float32)
                    + updb[...])

    jj = lax.broadcasted_iota(jnp.int32, (BS, B), 1)
    win_out[...] = jnp.max(jnp.where(ucol[...] == urow[...], jj, -1), axis=1,
                           keepdims=True)


def _tc_mlp(nf_rows, ef_rows, dt_flat, wflat, w2d, src_rows, ucol, urow,
            node_features, w1n, w1e, w1t, b1, w2, b2, w1s, b1s, w2s, b2s,
            updA, updB, updb, twp, tbp):
    full = lambda a: pl.BlockSpec(a.shape, lambda g: (0,) * a.ndim)
    return pl.pallas_call(
        _tc_mlp_body,
        grid=(GRID,),
        in_specs=[
            pl.BlockSpec((BS * K, D), lambda g: (g, 0)),
            pl.BlockSpec((BS * K, D_EDGE), lambda g: (g, 0)),
            pl.BlockSpec((BS * K, 1), lambda g: (g, 0)),
            pl.BlockSpec((BS * K, 1), lambda g: (g, 0)),
            pl.BlockSpec((BS, K), lambda g: (g, 0)),
            pl.BlockSpec((BS, D), lambda g: (g, 0)),
            pl.BlockSpec((BS, 1), lambda g: (g, 0)),
            pl.BlockSpec((1, B), lambda g: (0, 0)),
            pl.BlockSpec((TBLK, D), lambda g: (g, 0)),
        ] + [full(a) for a in (w1n, w1e, w1t, b1, w2, b2, w1s, b1s, w2s, b2s,
                               updA, updB, updb, twp, tbp)],
        out_specs=[
            pl.BlockSpec((BS, D), lambda g: (g, 0)),
            pl.BlockSpec((BS, 1), lambda g: (g, 0)),
            pl.BlockSpec((TBLK, D), lambda g: (g, 0)),
        ],
        out_shape=[
            jax.ShapeDtypeStruct((B, D), jnp.float32),
            jax.ShapeDtypeStruct((B, 1), jnp.int32),
            jax.ShapeDtypeStruct((N_NODES, D), jnp.float32),
        ],
        name="tc_mlp",
    )(nf_rows, ef_rows, dt_flat, wflat, w2d, src_rows, ucol, urow,
      node_features, w1n, w1e, w1t, b1, w2, b2, w1s, b1s, w2s, b2s,
      updA, updB, updb, twp, tbp)


def _sc_scatter_body(tbl, emb, u, win, u0, u1, wv0, wv1, r0, r1, s0, s1):
    wid = lax.axis_index("s") * NC + lax.axis_index("c")
    sbase = wid * SPW
    pltpu.sync_copy(u.at[pl.ds(sbase, CH)], u0)
    pltpu.sync_copy(u.at[pl.ds(sbase + CH, CH)], u1)
    pltpu.sync_copy(win.at[pl.ds(sbase, CH)], wv0)
    pltpu.sync_copy(win.at[pl.ds(sbase + CH, CH)], wv1)
    g0 = pltpu.async_copy(emb.at[wv0], r0, s0)
    g1 = pltpu.async_copy(emb.at[wv1], r1, s1)
    g0.wait()
    d0 = pltpu.async_copy(r0, tbl.at[u0], s0)
    g1.wait()
    d1 = pltpu.async_copy(r1, tbl.at[u1], s1)
    d0.wait()
    d1.wait()


def _sc_scatter(tbl_ref, emb, unique_nodes, winner):
    mesh = plsc.VectorSubcoreMesh(core_axis_name="c", subcore_axis_name="s")
    pl.kernel(
        _sc_scatter_body,
        out_type=(),
        mesh=mesh,
        scratch_types=(
            pltpu.VMEM((CH,), jnp.int32),
            pltpu.VMEM((CH,), jnp.int32),
            pltpu.VMEM((CH,), jnp.int32),
            pltpu.VMEM((CH,), jnp.int32),
            pltpu.VMEM((CH, D), jnp.float32),
            pltpu.VMEM((CH, D), jnp.float32),
            pltpu.SemaphoreType.DMA,
            pltpu.SemaphoreType.DMA,
        ),
        name="sc_scatter",
    )(tbl_ref, emb, unique_nodes, winner)


def kernel(unique_nodes, selected_node, selected_edge_idxs, selected_delta_time,
           selected_weight, node_features, edge_features, time_w, time_b,
           fc1_W, fc1_b, fc2_W, fc2_b, fc1s_W, fc1s_b, fc2s_W, fc2s_b,
           upd_W, upd_b):
    selnode_flat = selected_node.reshape(BK)
    seledge_flat = selected_edge_idxs.reshape(BK)
    dt_flat = selected_delta_time.reshape(BK, 1)
    wflat = selected_weight.reshape(BK, 1)
    ucol = unique_nodes.reshape(B, 1)
    urow = unique_nodes.reshape(1, B)

    w1n = fc1_W[:D]
    w1e = fc1_W[D:D + D_EDGE]
    w1t = jnp.zeros((128, D), jnp.float32).at[:D_TIME].set(fc1_W[D + D_EDGE:])
    twp = jnp.zeros((1, 128), jnp.float32).at[:, :D_TIME].set(time_w[None, :])
    tbp = jnp.zeros((1, 128), jnp.float32).at[:, :D_TIME].set(time_b[None, :])
    b1 = fc1_b.reshape(1, D)
    b2 = fc2_b.reshape(1, D)
    b1s = fc1s_b.reshape(1, D)
    b2s = fc2s_b.reshape(1, D)
    updA = upd_W[:D]
    updB = upd_W[D:]
    updb = upd_b.reshape(1, D)

    nf_rows, src_rows = _sc_gather(node_features, selnode_flat, unique_nodes)
    ef_rows = _sc_gather_edges(edge_features, seledge_flat)

    emb, winner, tbl_copy = _tc_mlp(
        nf_rows, ef_rows, dt_flat, wflat, selected_weight, src_rows, ucol,
        urow, node_features, w1n, w1e, w1t, b1, w2=fc2_W, b2=b2, w1s=fc1s_W,
        b1s=b1s, w2s=fc2s_W, b2s=b2s, updA=updA, updB=updB, updb=updb,
        twp=twp, tbp=tbp)

    tbl_ref = jax.new_ref(tbl_copy)
    _sc_scatter(tbl_ref, emb, unique_nodes, winner.reshape(B))
    return emb, tbl_ref[...]
